# Initial kernel scaffold; baseline (speedup 1.0000x reference)
import functools

import jax
import jax.numpy as jnp
from jax import lax
from jax.experimental import pallas as pl
from jax.experimental.pallas import tpu as pltpu
from jax.experimental.pallas import tpu_sc as plsc

_BATCH = 16384
_FIELDS = 100
_DIM = 32
_B = _BATCH * _FIELDS

_NC = 2
_NS = 16
_NW = _NC * _NS
_BPW = _B // _NW
_CHUNK = 1600
_NCHUNKS = _BPW // _CHUNK


def _make_kernel():
  mesh = plsc.VectorSubcoreMesh(core_axis_name="c", subcore_axis_name="s")

  @functools.partial(
      pl.kernel,
      out_type=jax.ShapeDtypeStruct((_B, _DIM), jnp.float32),
      mesh=mesh,
      scratch_types=[
          pltpu.VMEM((_CHUNK,), jnp.int32),
          pltpu.VMEM((_CHUNK, _DIM), jnp.float32),
          pltpu.SemaphoreType.DMA,
      ],
  )
  def gather_kernel(idx_hbm, table_hbm, out_hbm, idx_v, rows_v, sem):
    wid = lax.axis_index("s") * _NC + lax.axis_index("c")
    base = wid * _BPW

    def chunk_body(g, carry):
      off = base + g * _CHUNK
      pltpu.sync_copy(idx_hbm.at[pl.ds(off, _CHUNK)], idx_v)
      pltpu.async_copy(table_hbm.at[idx_v], rows_v, sem).wait()
      pltpu.sync_copy(rows_v, out_hbm.at[pl.ds(off, _CHUNK)])
      return carry

    lax.fori_loop(0, _NCHUNKS, chunk_body, 0)

  return gather_kernel


_kernel_fn = _make_kernel()


@jax.jit
def kernel(region_ids, embedding_table):
  ids_flat = region_ids.reshape(_B).astype(jnp.int32)
  out = _kernel_fn(ids_flat, embedding_table)
  return out.reshape(_BATCH, _FIELDS, _DIM)


# speedup vs baseline: 1.1063x; 1.1063x over previous
import functools

import jax
import jax.numpy as jnp
from jax import lax
from jax.experimental import pallas as pl
from jax.experimental.pallas import tpu as pltpu
from jax.experimental.pallas import tpu_sc as plsc

_BATCH = 16384
_FIELDS = 100
_DIM = 32
_B = _BATCH * _FIELDS

_NC = 2
_NS = 16
_NW = _NC * _NS
_BPW = _B // _NW
_CHUNK = 1600
_NCHUNKS = _BPW // _CHUNK


def _make_kernel():
  mesh = plsc.VectorSubcoreMesh(core_axis_name="c", subcore_axis_name="s")

  @functools.partial(
      pl.kernel,
      out_type=jax.ShapeDtypeStruct((_B, _DIM), jnp.float32),
      mesh=mesh,
      scratch_types=[
          pltpu.VMEM((_CHUNK,), jnp.int32),
          pltpu.VMEM((_CHUNK, _DIM), jnp.float32),
          pltpu.SemaphoreType.DMA,
      ],
      compiler_params=pltpu.CompilerParams(use_tc_tiling_on_sc=False),
  )
  def gather_kernel(idx_hbm, table_hbm, out_hbm, idx_v, rows_v, sem):
    wid = lax.axis_index("s") * _NC + lax.axis_index("c")
    base = wid * _BPW

    def chunk_body(g, carry):
      off = base + g * _CHUNK
      pltpu.sync_copy(idx_hbm.at[pl.ds(off, _CHUNK)], idx_v)
      pltpu.async_copy(table_hbm.at[idx_v], rows_v, sem).wait()
      pltpu.sync_copy(rows_v, out_hbm.at[pl.ds(off, _CHUNK)])
      return carry

    lax.fori_loop(0, _NCHUNKS, chunk_body, 0)

  return gather_kernel


_kernel_fn = _make_kernel()


@jax.jit
def kernel(region_ids, embedding_table):
  ids_flat = region_ids.reshape(_B).astype(jnp.int32)
  out = _kernel_fn(ids_flat, embedding_table)
  return out.reshape(_BATCH, _FIELDS, _DIM)


# speedup vs baseline: 1.1116x; 1.0048x over previous
import functools

import jax
import jax.numpy as jnp
from jax import lax
from jax.experimental import pallas as pl
from jax.experimental.pallas import tpu as pltpu
from jax.experimental.pallas import tpu_sc as plsc

_BATCH = 16384
_FIELDS = 100
_DIM = 32
_B = _BATCH * _FIELDS

_NC = 2
_NS = 16
_NW = _NC * _NS
_BPW = _B // _NW
_CHUNK = 1600
_NCHUNKS = _BPW // _CHUNK


def _make_kernel():
  mesh = plsc.VectorSubcoreMesh(core_axis_name="c", subcore_axis_name="s")

  @functools.partial(
      pl.kernel,
      out_type=jax.ShapeDtypeStruct((_B, _DIM), jnp.float32),
      mesh=mesh,
      scratch_types=[
          pltpu.VMEM((_CHUNK,), jnp.int32),
          pltpu.VMEM((_CHUNK,), jnp.int32),
          pltpu.VMEM((_CHUNK, _DIM), jnp.float32),
          pltpu.VMEM((_CHUNK, _DIM), jnp.float32),
          pltpu.SemaphoreType.DMA,
          pltpu.SemaphoreType.DMA,
          pltpu.SemaphoreType.DMA,
          pltpu.SemaphoreType.DMA,
          pltpu.SemaphoreType.DMA,
          pltpu.SemaphoreType.DMA,
      ],
      compiler_params=pltpu.CompilerParams(use_tc_tiling_on_sc=False),
  )
  def gather_kernel(idx_hbm, table_hbm, out_hbm,
                    idx_v0, idx_v1, rows_v0, rows_v1,
                    sem_i0, sem_i1, sem_g0, sem_g1, sem_o0, sem_o1):
    idx_v = (idx_v0, idx_v1)
    rows_v = (rows_v0, rows_v1)
    sem_i = (sem_i0, sem_i1)
    sem_g = (sem_g0, sem_g1)
    sem_o = (sem_o0, sem_o1)

    wid = lax.axis_index("s") * _NC + lax.axis_index("c")
    base = wid * _BPW

    def idx_copy(g, s):
      return pltpu.make_async_copy(
          idx_hbm.at[pl.ds(base + g * _CHUNK, _CHUNK)], idx_v[s], sem_i[s])

    def out_copy(g, s):
      return pltpu.make_async_copy(
          rows_v[s], out_hbm.at[pl.ds(base + g * _CHUNK, _CHUNK)], sem_o[s])

    idx_copy(0, 0).start()
    idx_copy(1, 1).start()

    def pair_body(h, carry):
      for s in (0, 1):
        g = h * 2 + s
        idx_copy(g, s).wait()

        @pl.when(h >= 1)
        def _wait_out():
          out_copy(g - 2, s).wait()

        gather = pltpu.make_async_copy(
            table_hbm.at[idx_v[s]], rows_v[s], sem_g[s])
        gather.start()
        gather.wait()

        out_copy(g, s).start()

        @pl.when(h < _NCHUNKS // 2 - 1)
        def _prefetch_idx():
          idx_copy(g + 2, s).start()
      return carry

    lax.fori_loop(0, _NCHUNKS // 2, pair_body, 0)

    out_copy(_NCHUNKS - 2, 0).wait()
    out_copy(_NCHUNKS - 1, 1).wait()

  return gather_kernel


_kernel_fn = _make_kernel()


@jax.jit
def kernel(region_ids, embedding_table):
  ids_flat = region_ids.reshape(_B).astype(jnp.int32)
  out = _kernel_fn(ids_flat, embedding_table)
  return out.reshape(_BATCH, _FIELDS, _DIM)


# speedup vs baseline: 4.4213x; 3.9773x over previous
import functools

import jax
import jax.numpy as jnp
from jax import lax
from jax.experimental import pallas as pl
from jax.experimental.pallas import tpu as pltpu
from jax.experimental.pallas import tpu_sc as plsc

_BATCH = 16384
_FIELDS = 100
_DIM = 32
_B = _BATCH * _FIELDS

_NC = 2
_NS = 16
_NW = _NC * _NS
_BPW = _B // _NW
_CHUNK = 1600
_NCHUNKS = _BPW // _CHUNK


def _make_kernel():
  mesh = plsc.VectorSubcoreMesh(core_axis_name="c", subcore_axis_name="s")

  @functools.partial(
      pl.kernel,
      out_type=jax.ShapeDtypeStruct((_BATCH, _FIELDS, _DIM), jnp.float32),
      mesh=mesh,
      scratch_types=[
          pltpu.VMEM((_CHUNK,), jnp.int32),
          pltpu.VMEM((_CHUNK,), jnp.int32),
          pltpu.VMEM((_CHUNK, _DIM), jnp.float32),
          pltpu.VMEM((_CHUNK, _DIM), jnp.float32),
          pltpu.SemaphoreType.DMA,
          pltpu.SemaphoreType.DMA,
          pltpu.SemaphoreType.DMA,
          pltpu.SemaphoreType.DMA,
          pltpu.SemaphoreType.DMA,
          pltpu.SemaphoreType.DMA,
      ],
      compiler_params=pltpu.CompilerParams(use_tc_tiling_on_sc=False),
  )
  def gather_kernel(idx_hbm, table_hbm, out_hbm,
                    idx_v0, idx_v1, rows_v0, rows_v1,
                    sem_i0, sem_i1, sem_g0, sem_g1, sem_o0, sem_o1):
    idx_v = (idx_v0, idx_v1)
    rows_v = (rows_v0, rows_v1)
    sem_i = (sem_i0, sem_i1)
    sem_g = (sem_g0, sem_g1)
    sem_o = (sem_o0, sem_o1)

    wid = lax.axis_index("s") * _NC + lax.axis_index("c")
    base = wid * _BPW

    def idx_copy(g, s):
      return pltpu.make_async_copy(
          idx_hbm.at[pl.ds(base + g * _CHUNK, _CHUNK)], idx_v[s], sem_i[s])

    bbase = wid * (_BPW // _FIELDS)
    _CBATCH = _CHUNK // _FIELDS

    def out_copies(g, s):
      b0 = bbase + g * _CBATCH
      return [
          pltpu.make_async_copy(
              rows_v[s].at[pl.ds(j * _FIELDS, _FIELDS)],
              out_hbm.at[b0 + j], sem_o[s])
          for j in range(_CBATCH)
      ]

    idx_copy(0, 0).start()
    idx_copy(1, 1).start()

    def pair_body(h, carry):
      for s in (0, 1):
        g = h * 2 + s
        idx_copy(g, s).wait()

        @pl.when(h >= 1)
        def _wait_out():
          for c in out_copies(g - 2, s):
            c.wait()

        gather = pltpu.make_async_copy(
            table_hbm.at[idx_v[s]], rows_v[s], sem_g[s])
        gather.start()
        gather.wait()

        for c in out_copies(g, s):
          c.start()

        @pl.when(h < _NCHUNKS // 2 - 1)
        def _prefetch_idx():
          idx_copy(g + 2, s).start()
      return carry

    lax.fori_loop(0, _NCHUNKS // 2, pair_body, 0)

    for c in out_copies(_NCHUNKS - 2, 0):
      c.wait()
    for c in out_copies(_NCHUNKS - 1, 1):
      c.wait()

  return gather_kernel


_kernel_fn = _make_kernel()


@jax.jit
def kernel(region_ids, embedding_table):
  return _kernel_fn(region_ids.reshape(_B), embedding_table)


# speedup vs baseline: 6.5039x; 1.4711x over previous
import functools

import jax
import jax.numpy as jnp
from jax import lax
from jax.experimental import pallas as pl
from jax.experimental.pallas import tpu as pltpu
from jax.experimental.pallas import tpu_sc as plsc

_BATCH = 16384
_FIELDS = 100
_DIM = 32
_B = _BATCH * _FIELDS

_NC = 2
_NS = 16
_NW = _NC * _NS
_BPW = _B // _NW
_CHUNK = 1600
_NCHUNKS = _BPW // _CHUNK


def _make_kernel():
  mesh = plsc.VectorSubcoreMesh(core_axis_name="c", subcore_axis_name="s")

  @functools.partial(
      pl.kernel,
      out_type=jax.ShapeDtypeStruct((_B, _DIM), jnp.float32),
      mesh=mesh,
      scratch_types=[
          pltpu.VMEM((_CHUNK,), jnp.int32),
          pltpu.VMEM((_CHUNK,), jnp.int32),
          pltpu.VMEM((_CHUNK, _DIM), jnp.float32),
          pltpu.VMEM((_CHUNK, _DIM), jnp.float32),
          pltpu.SemaphoreType.DMA,
          pltpu.SemaphoreType.DMA,
          pltpu.SemaphoreType.DMA,
          pltpu.SemaphoreType.DMA,
          pltpu.SemaphoreType.DMA,
          pltpu.SemaphoreType.DMA,
      ],
      compiler_params=pltpu.CompilerParams(use_tc_tiling_on_sc=False),
  )
  def gather_kernel(idx_hbm, table_hbm, out_hbm,
                    idx_v0, idx_v1, rows_v0, rows_v1,
                    sem_i0, sem_i1, sem_g0, sem_g1, sem_o0, sem_o1):
    idx_v = (idx_v0, idx_v1)
    rows_v = (rows_v0, rows_v1)
    sem_i = (sem_i0, sem_i1)
    sem_g = (sem_g0, sem_g1)
    sem_o = (sem_o0, sem_o1)

    wid = lax.axis_index("s") * _NC + lax.axis_index("c")
    base = wid * _BPW

    def idx_copy(g, s):
      return pltpu.make_async_copy(
          idx_hbm.at[pl.ds(base + g * _CHUNK, _CHUNK)], idx_v[s], sem_i[s])

    def out_copies(g, s):
      return [
          pltpu.make_async_copy(
              rows_v[s],
              out_hbm.at[pl.ds(base + g * _CHUNK, _CHUNK)], sem_o[s])
      ]

    idx_copy(0, 0).start()
    idx_copy(1, 1).start()

    def pair_body(h, carry):
      for s in (0, 1):
        g = h * 2 + s
        idx_copy(g, s).wait()

        @pl.when(h >= 1)
        def _wait_out():
          for c in out_copies(g - 2, s):
            c.wait()

        gather = pltpu.make_async_copy(
            table_hbm.at[idx_v[s]], rows_v[s], sem_g[s])
        gather.start()
        gather.wait()

        for c in out_copies(g, s):
          c.start()

        @pl.when(h < _NCHUNKS // 2 - 1)
        def _prefetch_idx():
          idx_copy(g + 2, s).start()
      return carry

    lax.fori_loop(0, _NCHUNKS // 2, pair_body, 0)

    for c in out_copies(_NCHUNKS - 2, 0):
      c.wait()
    for c in out_copies(_NCHUNKS - 1, 1):
      c.wait()

  return gather_kernel


_kernel_fn = _make_kernel()

_BBLK = 128


def _tc_transpose_body(x_ref, o_ref):
  x = x_ref[...]
  for k in range(_FIELDS * _DIM // _BBLK):
    t = x[:, k * _BBLK:(k + 1) * _BBLK].T
    for j in range(4):
      f = k * 4 + j
      o_ref[f] = t[j * _DIM:(j + 1) * _DIM, :]


_tc_transpose = pl.pallas_call(
    _tc_transpose_body,
    grid=(_BATCH // _BBLK,),
    in_specs=[pl.BlockSpec((_BBLK, _FIELDS * _DIM), lambda i: (i, 0))],
    out_specs=pl.BlockSpec((_FIELDS, _DIM, _BBLK), lambda i: (0, 0, i)),
    out_shape=jax.ShapeDtypeStruct((_FIELDS, _DIM, _BATCH), jnp.float32),
)


@jax.jit
def kernel(region_ids, embedding_table):
  rows = _kernel_fn(region_ids.reshape(_B), embedding_table)
  out = _tc_transpose(rows.reshape(_BATCH, _FIELDS * _DIM))
  return out.transpose(2, 0, 1)
